# baseline (speedup 1.0000x reference)
import functools

import jax
import jax.numpy as jnp
from jax import lax
from jax.experimental import pallas as pl
from jax.experimental.pallas import tpu as pltpu
from jax.experimental.pallas import tpu_sc as plsc

MODENUM = 2
NODENUM = 10000
SLID = 64
EMB = 4
GRU_H = 16
E = 640000
HID = 32 * 6

NS = 16
NROWS = 10240
ROWS_PT = NROWS // NS
TROWS = 20480
TROWS_PT = TROWS // NS
EPT = E // NS
KCH = 800
NCHUNKS = EPT // KCH


def _make_sc_pass(ncol, gather):
    mesh = plsc.VectorSubcoreMesh(core_axis_name="c", subcore_axis_name="s")
    scratch = [pltpu.VMEM((KCH,), jnp.int32),
               pltpu.VMEM((KCH,), jnp.int32),
               pltpu.VMEM((KCH,), jnp.float32)]
    scratch += [pltpu.VMEM((KCH, 16), jnp.float32) for _ in range(ncol)]
    scratch += [pltpu.VMEM_SHARED((NROWS, 16), jnp.float32)
                for _ in range(ncol)]
    if gather:
        scratch += [pltpu.VMEM_SHARED((NROWS, 16), jnp.float32)
                    for _ in range(ncol)]
    scratch += [pltpu.SemaphoreType.DMA]

    @functools.partial(
        pl.kernel, mesh=mesh,
        out_type=jax.ShapeDtypeStruct((MODENUM, ncol, NROWS, 16),
                                      jnp.float32),
        scratch_types=scratch,
        compiler_params=pltpu.CompilerParams(use_tc_tiling_on_sc=False))
    def kfn(src_h, dst_h, wei_h, *tab_out_scratch):
        tabs = tab_out_scratch[:ncol] if gather else ()
        rest = tab_out_scratch[ncol:] if gather else tab_out_scratch
        out_h = rest[0]
        idx_s, idx_d, wv = rest[1:4]
        rows = rest[4:4 + ncol]
        accs = rest[4 + ncol:4 + 2 * ncol]
        tsp = rest[4 + 2 * ncol:4 + 3 * ncol] if gather else ()
        sem = rest[-1]
        c = lax.axis_index("c")
        s = lax.axis_index("s")

        zero16 = jnp.zeros((16,), jnp.float32)

        def zero_row(i, carry):
            for r in rows:
                r[i] = zero16
            return carry

        lax.fori_loop(0, KCH, zero_row, 0)
        for j in range(ncol):
            pltpu.sync_copy(rows[j].at[pl.ds(0, ROWS_PT)],
                            accs[j].at[pl.ds(s * ROWS_PT, ROWS_PT)])
            if gather:
                pltpu.sync_copy(
                    tabs[j].at[pl.ds(c * NROWS + s * ROWS_PT, ROWS_PT)],
                    tsp[j].at[pl.ds(s * ROWS_PT, ROWS_PT)])
        plsc.subcore_barrier()

        one0 = jnp.where(lax.iota(jnp.int32, 16) == 0, 1.0, 0.0)

        def chunk(g, carry):
            base = pl.multiple_of(c * E + s * EPT + g * KCH, 8)
            pltpu.sync_copy(dst_h.at[pl.ds(base, KCH)], idx_d)
            pltpu.sync_copy(wei_h.at[pl.ds(base, KCH)], wv)
            if gather:
                pltpu.sync_copy(src_h.at[pl.ds(base, KCH)], idx_s)
                for j in range(ncol):
                    pltpu.async_copy(tsp[j].at[idx_s], rows[j], sem).wait()

            def scale(gg, cr):
                wvec = wv[pl.ds(gg * 16, 16)]
                for i in range(16):
                    k = gg * 16 + i
                    wbc = jnp.broadcast_to(wvec[i:i + 1], (16,))
                    if gather:
                        for j in range(ncol):
                            rows[j][k] = rows[j][k] * wbc
                    else:
                        rows[0][k] = wbc * one0
                return cr

            lax.fori_loop(0, KCH // 16, scale, 0)
            for j in range(ncol):
                pltpu.sync_copy(rows[j], accs[j].at[idx_d], add=True)
            return carry

        lax.fori_loop(0, NCHUNKS, chunk, 0)
        plsc.subcore_barrier()
        for j in range(ncol):
            pltpu.sync_copy(accs[j].at[pl.ds(s * ROWS_PT, ROWS_PT)],
                            out_h.at[c, j, pl.ds(s * ROWS_PT, ROWS_PT)])

    return kfn


_make_sc_pass_cached = functools.lru_cache(maxsize=None)(_make_sc_pass)


def _run_sc_pass(src, dst, wei, table, ncol, gather):
    fn = _make_sc_pass_cached(ncol, gather)
    if gather:
        tm = table.reshape(MODENUM, NODENUM, table.shape[1])
        tp = jnp.pad(tm, ((0, 0), (0, NROWS - NODENUM), (0, 0))).reshape(
            MODENUM * NROWS, table.shape[1])
        tabs = [tp[:, 16 * j:16 * (j + 1)] for j in range(ncol)]
        out = fn(src, dst, wei, *tabs)
    else:
        out = fn(src, dst, wei)
    return out.transpose(0, 2, 1, 3).reshape(
        MODENUM, NROWS, 16 * ncol)[:, :NODENUM, :]


def _sc_deg(src, dst, wei, table):
    return _run_sc_pass(src, dst, wei, table, 1, False)


def _sc_pass16(src, dst, wei, table):
    return _run_sc_pass(src, dst, wei, table, 1, True)


def _sc_pass64(src, dst, wei, table):
    lo = _run_sc_pass(src, dst, wei, table[:, 0:32], 2, True)
    hi = _run_sc_pass(src, dst, wei, table[:, 32:64], 2, True)
    return jnp.concatenate([lo, hi], axis=2)


ROWB = 2000


def _row_call(body, ins, in_dims, out_shapes, out_dims):
    n_grid = NODENUM // ROWB

    def spec(shape, dim):
        if dim is None:
            return pl.BlockSpec(shape, lambda i: (0,) * len(shape))
        blk = tuple(ROWB if k == dim else shape[k] for k in range(len(shape)))

        def idx(i, d=dim, r=len(shape)):
            return tuple(i if k == d else 0 for k in range(r))

        return pl.BlockSpec(blk, idx)

    return pl.pallas_call(
        body,
        grid=(n_grid,),
        in_specs=[spec(x.shape, d) for x, d in zip(ins, in_dims)],
        out_specs=tuple(spec(s.shape, d)
                        for s, d in zip(out_shapes, out_dims)),
        out_shape=tuple(out_shapes),
    )(*ins)


def _kp_body(g1W, g2W, g1b, r1W, r2W, r1b, W14, c1, rWW, c1r):
    for m in range(MODENUM):
        W14[m] = jnp.dot(g1W[m], g2W[m], preferred_element_type=jnp.float32)
        c1[m] = jnp.dot(g1b[m], g2W[m], preferred_element_type=jnp.float32)
        rWW[m] = jnp.dot(r1W[m], r2W[m], preferred_element_type=jnp.float32)
        c1r[m] = jnp.dot(r1b[m], r2W[m], preferred_element_type=jnp.float32)


def _k1_body(xt, W14, nfW1, nfb1, nfW2, nfb2, fwdtab_un, lin):
    nb = xt.shape[1]
    ones = jnp.ones((nb, 1), jnp.float32)
    zeros = jnp.zeros((nb, 11), jnp.float32)
    for m in range(MODENUM):
        u = jnp.dot(xt[m], W14[m], preferred_element_type=jnp.float32)
        fwdtab_un[m] = jnp.concatenate([u, ones, zeros], axis=1)
        h = jnp.maximum(
            jnp.dot(xt[m], nfW1[...], preferred_element_type=jnp.float32)
            + nfb1[...], 0.0)
        lin[m] = (jnp.dot(h, nfW2[...], preferred_element_type=jnp.float32)
                  + nfb2[...])


def _k2_body(deg, fwdtab_un, dinv, fwdtab):
    degf = deg[...] + 1.0
    di = jnp.where(degf > 0, lax.rsqrt(degf), 0.0)
    dinv[...] = di
    fwdtab[...] = fwdtab_un[...] * di


def _k3_body(acc1, fwdtab, dinv, table2, rs):
    out1 = dinv[...] * (acc1[...] + fwdtab[...])
    table2[...] = dinv[...] * out1
    rs[...] = out1[:, :, 4:5]


def _k4_body(acc2, table2, dinv, rs, c1, g2b, lin, cat,
             mlW1, mlb1, mlW2, mlb2, rWW, rnW1, rnb1, rnW2, rnb2,
             seq, rectab1, rl):
    out2 = dinv[...] * (acc2[...] + table2[...])
    nb = out2.shape[1]
    sq = jnp.zeros((nb, 2 * EMB), jnp.float32)
    for m in range(MODENUM):
        g2 = out2[m, :, 0:EMB] + rs[m] * c1[m] + g2b[m]
        cc = jnp.dot(jnp.concatenate([g2, lin[m]], axis=1), cat[m],
                     preferred_element_type=jnp.float32)
        sq = sq + cc
    seq[...] = sq
    h = jnp.maximum(
        jnp.dot(sq, mlW1[...], preferred_element_type=jnp.float32)
        + mlb1[...], 0.0)
    re_line = (jnp.dot(h, mlW2[...], preferred_element_type=jnp.float32)
               + mlb2[...])
    for m in range(MODENUM):
        rectab1[m] = dinv[m] * jnp.dot(re_line, rWW[m],
                                       preferred_element_type=jnp.float32)
    h2 = jnp.maximum(
        jnp.dot(re_line, rnW1[...], preferred_element_type=jnp.float32)
        + rnb1[...], 0.0)
    rl[...] = (jnp.dot(h2, rnW2[...], preferred_element_type=jnp.float32)
               + rnb2[...])


def _k5_body(accr1, rectab1, dinv, rectab2):
    di = dinv[...]
    rectab2[...] = di * di * (accr1[...] + rectab1[...])


def _k6_body(accr2, rectab2, dinv, rs, c1r, r2b, rl, recW, recb, rec_res):
    nb = rl.shape[0]
    rsum = jnp.zeros((nb, SLID), jnp.float32)
    for m in range(MODENUM):
        r = dinv[m] * (accr2[m] + rectab2[m]) + rs[m] * c1r[m] + r2b[m]
        rsum = rsum + r
    rec_res[...] = (
        jnp.dot(rsum, recW[0:SLID, :], preferred_element_type=jnp.float32)
        + 2.0 * jnp.dot(rl[...], recW[SLID:2 * SLID, :],
                        preferred_element_type=jnp.float32)
        + recb[...])


_GRU_BLK = 8


def _gru_scan(gi_ref, whht_ref, bhh, h_init, yout_ref, T):
    def blk(b, h):
        gi = gi_ref[pl.ds(b * _GRU_BLK, _GRU_BLK), :]
        ys = []
        for i in range(_GRU_BLK):
            g = gi[i:i + 1, :]
            gh = jnp.dot(h, whht_ref[...],
                         preferred_element_type=jnp.float32) + bhh
            r = jax.nn.sigmoid(g[:, 0:16] + gh[:, 0:16])
            z = jax.nn.sigmoid(g[:, 16:32] + gh[:, 16:32])
            n = jnp.tanh(g[:, 32:48] + r * gh[:, 32:48])
            h = (1.0 - z) * n + z * h
            ys.append(h)
        yout_ref[pl.ds(b * _GRU_BLK, _GRU_BLK), :] = jnp.concatenate(
            ys, axis=0)
        return h

    return lax.fori_loop(0, T // _GRU_BLK, blk, h_init)


def _gru_body(seq_ref, h0_ref,
              wih0t_ref, whh0t_ref, b0_ref,
              wih1t_ref, whh1t_ref, b1_ref,
              y_ref, hT_ref, gi_ref, y0_ref):
    T = seq_ref.shape[0]
    gi_ref[...] = (
        jnp.dot(seq_ref[...], wih0t_ref[...],
                preferred_element_type=jnp.float32) + b0_ref[0:1, :])
    h0f = _gru_scan(gi_ref, whh0t_ref, b0_ref[1:2, :], h0_ref[0:1, :],
                    y0_ref, T)
    gi_ref[...] = (
        jnp.dot(y0_ref[...], wih1t_ref[...],
                preferred_element_type=jnp.float32) + b1_ref[0:1, :])
    h1f = _gru_scan(gi_ref, whh1t_ref, b1_ref[1:2, :], h0_ref[1:2, :],
                    y_ref, T)
    hT_ref[0:1, :] = h0f
    hT_ref[1:2, :] = h1f


_CF_KB = 1280


def _cf1_body(flat, w1, out, acc):
    @pl.when(pl.program_id(0) == 0)
    def _():
        acc[...] = jnp.zeros_like(acc)

    acc[...] += jnp.dot(flat[...], w1[...], preferred_element_type=jnp.float32)

    @pl.when(pl.program_id(0) == pl.num_programs(0) - 1)
    def _():
        out[...] = acc[...]


def _cf2_body(v, b1, w2, b2, w3, b3, out):
    h = jnp.maximum(v[...] + b1[...], 0.0)
    h = jnp.maximum(
        jnp.dot(h, w2[...], preferred_element_type=jnp.float32) + b2[...], 0.0)
    out[...] = (jnp.dot(h, w3[...], preferred_element_type=jnp.float32)
                + b3[...])


def kernel(x, wei, H_, params, adj):
    p = params
    f32 = jnp.float32
    xt = x.reshape(MODENUM, NODENUM, SLID)
    g1W = jnp.stack([p['g1W0'], p['g1W1']])
    g2W = jnp.stack([p['g2W0'], p['g2W1']])
    g1b = jnp.stack([p['g1b0'], p['g1b1']])[:, None, :]
    g2b = jnp.stack([p['g2b0'], p['g2b1']])[:, None, :]
    r1W = jnp.stack([p['r1W0'], p['r1W1']])
    r2W = jnp.stack([p['r2W0'], p['r2W1']])
    r1b = jnp.stack([p['r1b0'], p['r1b1']])[:, None, :]
    r2b = jnp.stack([p['r2b0'], p['r2b1']])[:, None, :]
    cat = jnp.stack([p['cat0'], p['cat1']])

    W14, c1, rWW, c1r = pl.pallas_call(
        _kp_body,
        out_shape=(jax.ShapeDtypeStruct((MODENUM, SLID, EMB), f32),
                   jax.ShapeDtypeStruct((MODENUM, 1, EMB), f32),
                   jax.ShapeDtypeStruct((MODENUM, SLID, SLID), f32),
                   jax.ShapeDtypeStruct((MODENUM, 1, SLID), f32)),
    )(g1W, g2W, g1b, r1W, r2W, r1b)

    fwdtab_un, lin = _row_call(
        _k1_body,
        [xt, W14, p['nfW1'], p['nfb1'][None, :], p['nfW2'],
         p['nfb2'][None, :]],
        [1, None, None, None, None, None],
        [jax.ShapeDtypeStruct((MODENUM, NODENUM, 16), f32),
         jax.ShapeDtypeStruct((MODENUM, NODENUM, EMB), f32)],
        [1, 1])

    src_f = adj[:, 0, :].reshape(MODENUM * E)
    dst_f = adj[:, 1, :].reshape(MODENUM * E)
    wei_f = wei.reshape(MODENUM * E)

    dummy_tab = jnp.zeros((8, 16), f32)
    deg16 = _sc_deg(src_f, dst_f, wei_f, dummy_tab)
    deg = deg16[:, :, 0:1]

    dinv, fwdtab = _row_call(
        _k2_body,
        [deg, fwdtab_un],
        [1, 1],
        [jax.ShapeDtypeStruct((MODENUM, NODENUM, 1), f32),
         jax.ShapeDtypeStruct((MODENUM, NODENUM, 16), f32)],
        [1, 1])

    acc1 = _sc_pass16(src_f, dst_f, wei_f,
                      fwdtab.reshape(MODENUM * NODENUM, 16))

    table2, rs = _row_call(
        _k3_body,
        [acc1, fwdtab, dinv],
        [1, 1, 1],
        [jax.ShapeDtypeStruct((MODENUM, NODENUM, 16), f32),
         jax.ShapeDtypeStruct((MODENUM, NODENUM, 1), f32)],
        [1, 1])

    acc2 = _sc_pass16(src_f, dst_f, wei_f,
                      table2.reshape(MODENUM * NODENUM, 16))

    seq, rectab1, rl = _row_call(
        _k4_body,
        [acc2, table2, dinv, rs, c1, g2b, lin, cat,
         p['mlW1'], p['mlb1'][None, :], p['mlW2'], p['mlb2'][None, :],
         rWW, p['rnW1'], p['rnb1'][None, :], p['rnW2'], p['rnb2'][None, :]],
        [1, 1, 1, 1, None, None, 1, None,
         None, None, None, None, None, None, None, None, None],
        [jax.ShapeDtypeStruct((NODENUM, 2 * EMB), f32),
         jax.ShapeDtypeStruct((MODENUM, NODENUM, SLID), f32),
         jax.ShapeDtypeStruct((NODENUM, SLID), f32)],
        [0, 1, 0])

    h0 = jnp.stack([H_[0, 0], H_[1, 0]])
    b0 = jnp.stack([p['bih0'], p['bhh0']])
    b1 = jnp.stack([p['bih1'], p['bhh1']])
    y1, hT = pl.pallas_call(
        _gru_body,
        out_shape=(jax.ShapeDtypeStruct((NODENUM, GRU_H), f32),
                   jax.ShapeDtypeStruct((2, GRU_H), f32)),
        scratch_shapes=[pltpu.VMEM((NODENUM, 3 * GRU_H), f32),
                        pltpu.VMEM((NODENUM, GRU_H), f32)],
    )(seq, h0, p['Wih0'].T, p['Whh0'].T, b0, p['Wih1'].T, p['Whh1'].T, b1)

    accr1 = _sc_pass64(src_f, dst_f, wei_f,
                       rectab1.reshape(MODENUM * NODENUM, SLID))

    rectab2, = _row_call(
        _k5_body,
        [accr1, rectab1, dinv],
        [1, 1, 1],
        [jax.ShapeDtypeStruct((MODENUM, NODENUM, SLID), f32)],
        [1])

    accr2 = _sc_pass64(src_f, dst_f, wei_f,
                       rectab2.reshape(MODENUM * NODENUM, SLID))

    rec_res, = _row_call(
        _k6_body,
        [accr2, rectab2, dinv, rs, c1r, r2b, rl, p['recW'],
         p['recb'][None, :]],
        [1, 1, 1, 1, None, None, 0, None, None],
        [jax.ShapeDtypeStruct((NODENUM, SLID), f32)],
        [0])

    flat = y1.reshape(1, NODENUM * GRU_H)
    nkb = (NODENUM * GRU_H) // _CF_KB
    v = pl.pallas_call(
        _cf1_body,
        grid=(nkb,),
        in_specs=[pl.BlockSpec((1, _CF_KB), lambda i: (0, i)),
                  pl.BlockSpec((_CF_KB, 256), lambda i: (i, 0))],
        out_specs=pl.BlockSpec((1, 256), lambda i: (0, 0)),
        out_shape=jax.ShapeDtypeStruct((1, 256), f32),
        scratch_shapes=[pltpu.VMEM((1, 256), f32)],
    )(flat, p['cfW1'])

    cf_res = pl.pallas_call(
        _cf2_body,
        out_shape=jax.ShapeDtypeStruct((1, 2), f32),
    )(v, p['cfb1'][None, :], p['cfW2'], p['cfb2'][None, :],
      p['cfW3'], p['cfb3'][None, :])

    new_H = hT[:, None, :]
    return cf_res, rec_res, new_H


# speedup vs baseline: 1.1856x; 1.1856x over previous
import functools

import jax
import jax.numpy as jnp
from jax import lax
from jax.experimental import pallas as pl
from jax.experimental.pallas import tpu as pltpu
from jax.experimental.pallas import tpu_sc as plsc

MODENUM = 2
NODENUM = 10000
SLID = 64
EMB = 4
GRU_H = 16
E = 640000
HID = 32 * 6

NS = 16
NROWS = 10240
ROWS_PT = NROWS // NS
TROWS = 20480
TROWS_PT = TROWS // NS
EPT = E // NS
KCH = 800
NCHUNKS = EPT // KCH


def _make_sc_pass(ncol, gather):
    mesh = plsc.VectorSubcoreMesh(core_axis_name="c", subcore_axis_name="s")
    scratch = [pltpu.VMEM((KCH,), jnp.int32),
               pltpu.VMEM((KCH,), jnp.int32),
               pltpu.VMEM((KCH,), jnp.float32)]
    scratch += [pltpu.VMEM((KCH, 16), jnp.float32) for _ in range(ncol)]
    scratch += [pltpu.VMEM_SHARED((NROWS, 16), jnp.float32)
                for _ in range(ncol)]
    if gather:
        scratch += [pltpu.VMEM_SHARED((NROWS, 16), jnp.float32)
                    for _ in range(ncol)]
    scratch += [pltpu.SemaphoreType.DMA]

    @functools.partial(
        pl.kernel, mesh=mesh,
        out_type=jax.ShapeDtypeStruct((MODENUM, ncol, NROWS, 16),
                                      jnp.float32),
        scratch_types=scratch,
        compiler_params=pltpu.CompilerParams(use_tc_tiling_on_sc=False))
    def kfn(src_h, dst_h, wei_h, *tab_out_scratch):
        tabs = tab_out_scratch[:ncol] if gather else ()
        rest = tab_out_scratch[ncol:] if gather else tab_out_scratch
        out_h = rest[0]
        idx_s, idx_d, wv = rest[1:4]
        rows = rest[4:4 + ncol]
        accs = rest[4 + ncol:4 + 2 * ncol]
        tsp = rest[4 + 2 * ncol:4 + 3 * ncol] if gather else ()
        sem = rest[-1]
        c = lax.axis_index("c")
        s = lax.axis_index("s")

        zero16 = jnp.zeros((16,), jnp.float32)

        def zero_row(i, carry):
            for r in rows:
                r[i] = zero16
            return carry

        lax.fori_loop(0, KCH, zero_row, 0)
        for j in range(ncol):
            pltpu.sync_copy(rows[j].at[pl.ds(0, ROWS_PT)],
                            accs[j].at[pl.ds(s * ROWS_PT, ROWS_PT)])
            if gather:
                pltpu.sync_copy(
                    tabs[j].at[pl.ds(c * NROWS + s * ROWS_PT, ROWS_PT)],
                    tsp[j].at[pl.ds(s * ROWS_PT, ROWS_PT)])
        plsc.subcore_barrier()

        one0 = jnp.where(lax.iota(jnp.int32, 16) == 0, 1.0, 0.0)

        def chunk(g, carry):
            base = pl.multiple_of(c * E + s * EPT + g * KCH, 8)
            pltpu.sync_copy(dst_h.at[pl.ds(base, KCH)], idx_d)
            pltpu.sync_copy(wei_h.at[pl.ds(base, KCH)], wv)
            if gather:
                pltpu.sync_copy(src_h.at[pl.ds(base, KCH)], idx_s)
                for j in range(ncol):
                    pltpu.async_copy(tsp[j].at[idx_s], rows[j], sem).wait()

            def scale(gg, cr):
                wvec = wv[pl.ds(gg * 16, 16)]
                for i in range(16):
                    k = gg * 16 + i
                    wbc = jnp.broadcast_to(wvec[i:i + 1], (16,))
                    if gather:
                        for j in range(ncol):
                            rows[j][k] = rows[j][k] * wbc
                    else:
                        rows[0][k] = wbc * one0
                return cr

            lax.fori_loop(0, KCH // 16, scale, 0)
            for j in range(ncol):
                pltpu.sync_copy(rows[j], accs[j].at[idx_d], add=True)
            return carry

        lax.fori_loop(0, NCHUNKS, chunk, 0)
        plsc.subcore_barrier()
        for j in range(ncol):
            pltpu.sync_copy(accs[j].at[pl.ds(s * ROWS_PT, ROWS_PT)],
                            out_h.at[c, j, pl.ds(s * ROWS_PT, ROWS_PT)])

    return kfn


_make_sc_pass_cached = functools.lru_cache(maxsize=None)(_make_sc_pass)


def _run_sc_pass(src, dst, wei, table, ncol, gather):
    fn = _make_sc_pass_cached(ncol, gather)
    if gather:
        tm = table.reshape(MODENUM, NODENUM, table.shape[1])
        tp = jnp.pad(tm, ((0, 0), (0, NROWS - NODENUM), (0, 0))).reshape(
            MODENUM * NROWS, table.shape[1])
        tabs = [tp[:, 16 * j:16 * (j + 1)] for j in range(ncol)]
        out = fn(src, dst, wei, *tabs)
    else:
        out = fn(src, dst, wei)
    return out.transpose(0, 2, 1, 3).reshape(
        MODENUM, NROWS, 16 * ncol)[:, :NODENUM, :]


def _sc_deg(src, dst, wei, table):
    return _run_sc_pass(src, dst, wei, table, 1, False)


def _sc_pass16(src, dst, wei, table):
    return _run_sc_pass(src, dst, wei, table, 1, True)


def _sc_pass64(src, dst, wei, table):
    lo = _run_sc_pass(src, dst, wei, table[:, 0:32], 2, True)
    hi = _run_sc_pass(src, dst, wei, table[:, 32:64], 2, True)
    return jnp.concatenate([lo, hi], axis=2)


ROWB = 2000


def _row_call(body, ins, in_dims, out_shapes, out_dims):
    n_grid = NODENUM // ROWB

    def spec(shape, dim):
        if dim is None:
            return pl.BlockSpec(shape, lambda i: (0,) * len(shape))
        blk = tuple(ROWB if k == dim else shape[k] for k in range(len(shape)))

        def idx(i, d=dim, r=len(shape)):
            return tuple(i if k == d else 0 for k in range(r))

        return pl.BlockSpec(blk, idx)

    return pl.pallas_call(
        body,
        grid=(n_grid,),
        in_specs=[spec(x.shape, d) for x, d in zip(ins, in_dims)],
        out_specs=tuple(spec(s.shape, d)
                        for s, d in zip(out_shapes, out_dims)),
        out_shape=tuple(out_shapes),
    )(*ins)


def _kp_body(g1W, g2W, g1b, r1W, r2W, r1b, W14, c1, rWW, c1r):
    for m in range(MODENUM):
        W14[m] = jnp.dot(g1W[m], g2W[m], preferred_element_type=jnp.float32)
        c1[m] = jnp.dot(g1b[m], g2W[m], preferred_element_type=jnp.float32)
        rWW[m] = jnp.dot(r1W[m], r2W[m], preferred_element_type=jnp.float32)
        c1r[m] = jnp.dot(r1b[m], r2W[m], preferred_element_type=jnp.float32)


def _k1_body(xt, W14, nfW1, nfb1, nfW2, nfb2, fwdtab_un, lin):
    nb = xt.shape[1]
    ones = jnp.ones((nb, 1), jnp.float32)
    zeros = jnp.zeros((nb, 11), jnp.float32)
    for m in range(MODENUM):
        u = jnp.dot(xt[m], W14[m], preferred_element_type=jnp.float32)
        fwdtab_un[m] = jnp.concatenate([u, ones, zeros], axis=1)
        h = jnp.maximum(
            jnp.dot(xt[m], nfW1[...], preferred_element_type=jnp.float32)
            + nfb1[...], 0.0)
        lin[m] = (jnp.dot(h, nfW2[...], preferred_element_type=jnp.float32)
                  + nfb2[...])


def _k2_body(deg, fwdtab_un, dinv, fwdtab):
    degf = deg[...] + 1.0
    di = jnp.where(degf > 0, lax.rsqrt(degf), 0.0)
    dinv[...] = di
    fwdtab[...] = fwdtab_un[...] * di


def _k3_body(acc1, fwdtab, dinv, table2, rs):
    out1 = dinv[...] * (acc1[...] + fwdtab[...])
    table2[...] = dinv[...] * out1
    rs[...] = out1[:, :, 4:5]


def _k4_body(acc2, table2, dinv, rs, c1, g2b, lin, cat,
             mlW1, mlb1, mlW2, mlb2, rWW, rnW1, rnb1, rnW2, rnb2,
             seq, rectab1, rl):
    out2 = dinv[...] * (acc2[...] + table2[...])
    nb = out2.shape[1]
    sq = jnp.zeros((nb, 2 * EMB), jnp.float32)
    for m in range(MODENUM):
        g2 = out2[m, :, 0:EMB] + rs[m] * c1[m] + g2b[m]
        cc = jnp.dot(jnp.concatenate([g2, lin[m]], axis=1), cat[m],
                     preferred_element_type=jnp.float32)
        sq = sq + cc
    seq[...] = sq
    h = jnp.maximum(
        jnp.dot(sq, mlW1[...], preferred_element_type=jnp.float32)
        + mlb1[...], 0.0)
    re_line = (jnp.dot(h, mlW2[...], preferred_element_type=jnp.float32)
               + mlb2[...])
    for m in range(MODENUM):
        rectab1[m] = dinv[m] * jnp.dot(re_line, rWW[m],
                                       preferred_element_type=jnp.float32)
    h2 = jnp.maximum(
        jnp.dot(re_line, rnW1[...], preferred_element_type=jnp.float32)
        + rnb1[...], 0.0)
    rl[...] = (jnp.dot(h2, rnW2[...], preferred_element_type=jnp.float32)
               + rnb2[...])


def _k5_body(accr1, rectab1, dinv, rectab2):
    di = dinv[...]
    rectab2[...] = di * di * (accr1[...] + rectab1[...])


def _k6_body(accr2, rectab2, dinv, rs, c1r, r2b, rl, recW, recb, rec_res):
    nb = rl.shape[0]
    rsum = jnp.zeros((nb, SLID), jnp.float32)
    for m in range(MODENUM):
        r = dinv[m] * (accr2[m] + rectab2[m]) + rs[m] * c1r[m] + r2b[m]
        rsum = rsum + r
    rec_res[...] = (
        jnp.dot(rsum, recW[0:SLID, :], preferred_element_type=jnp.float32)
        + 2.0 * jnp.dot(rl[...], recW[SLID:2 * SLID, :],
                        preferred_element_type=jnp.float32)
        + recb[...])


_GRU_BLK = 8


def _gru_scan(gi_ref, whht_ref, bhh, h_init, yout_ref, T):
    W = whht_ref[...]
    Wr = [W[k:k + 1, :] for k in range(GRU_H)]

    def blk(b, h):
        gi = gi_ref[pl.ds(b * _GRU_BLK, _GRU_BLK), :]
        ys = []
        for i in range(_GRU_BLK):
            g = gi[i:i + 1, :]
            parts = []
            for k0 in range(0, GRU_H, 4):
                t = jnp.broadcast_to(h[0:1, k0:k0 + 1], (1, 3 * GRU_H)) \
                    * Wr[k0]
                for k in range(k0 + 1, k0 + 4):
                    t = t + jnp.broadcast_to(h[0:1, k:k + 1],
                                             (1, 3 * GRU_H)) * Wr[k]
                parts.append(t)
            gh = ((parts[0] + parts[1]) + (parts[2] + parts[3])) + bhh
            r = jax.nn.sigmoid(g[:, 0:16] + gh[:, 0:16])
            z = jax.nn.sigmoid(g[:, 16:32] + gh[:, 16:32])
            n = jnp.tanh(g[:, 32:48] + r * gh[:, 32:48])
            h = (1.0 - z) * n + z * h
            ys.append(h)
        yout_ref[pl.ds(b * _GRU_BLK, _GRU_BLK), :] = jnp.concatenate(
            ys, axis=0)
        return h

    return lax.fori_loop(0, T // _GRU_BLK, blk, h_init)


def _gru_body(seq_ref, h0_ref,
              wih0t_ref, whh0t_ref, b0_ref,
              wih1t_ref, whh1t_ref, b1_ref,
              y_ref, hT_ref, gi_ref, y0_ref):
    T = seq_ref.shape[0]
    gi_ref[...] = (
        jnp.dot(seq_ref[...], wih0t_ref[...],
                preferred_element_type=jnp.float32) + b0_ref[0:1, :])
    h0f = _gru_scan(gi_ref, whh0t_ref, b0_ref[1:2, :], h0_ref[0:1, :],
                    y0_ref, T)
    gi_ref[...] = (
        jnp.dot(y0_ref[...], wih1t_ref[...],
                preferred_element_type=jnp.float32) + b1_ref[0:1, :])
    h1f = _gru_scan(gi_ref, whh1t_ref, b1_ref[1:2, :], h0_ref[1:2, :],
                    y_ref, T)
    hT_ref[0:1, :] = h0f
    hT_ref[1:2, :] = h1f


_CF_KB = 1280


def _cf1_body(flat, w1, out, acc):
    @pl.when(pl.program_id(0) == 0)
    def _():
        acc[...] = jnp.zeros_like(acc)

    acc[...] += jnp.dot(flat[...], w1[...], preferred_element_type=jnp.float32)

    @pl.when(pl.program_id(0) == pl.num_programs(0) - 1)
    def _():
        out[...] = acc[...]


def _cf2_body(v, b1, w2, b2, w3, b3, out):
    h = jnp.maximum(v[...] + b1[...], 0.0)
    h = jnp.maximum(
        jnp.dot(h, w2[...], preferred_element_type=jnp.float32) + b2[...], 0.0)
    out[...] = (jnp.dot(h, w3[...], preferred_element_type=jnp.float32)
                + b3[...])


def kernel(x, wei, H_, params, adj):
    p = params
    f32 = jnp.float32
    xt = x.reshape(MODENUM, NODENUM, SLID)
    g1W = jnp.stack([p['g1W0'], p['g1W1']])
    g2W = jnp.stack([p['g2W0'], p['g2W1']])
    g1b = jnp.stack([p['g1b0'], p['g1b1']])[:, None, :]
    g2b = jnp.stack([p['g2b0'], p['g2b1']])[:, None, :]
    r1W = jnp.stack([p['r1W0'], p['r1W1']])
    r2W = jnp.stack([p['r2W0'], p['r2W1']])
    r1b = jnp.stack([p['r1b0'], p['r1b1']])[:, None, :]
    r2b = jnp.stack([p['r2b0'], p['r2b1']])[:, None, :]
    cat = jnp.stack([p['cat0'], p['cat1']])

    W14, c1, rWW, c1r = pl.pallas_call(
        _kp_body,
        out_shape=(jax.ShapeDtypeStruct((MODENUM, SLID, EMB), f32),
                   jax.ShapeDtypeStruct((MODENUM, 1, EMB), f32),
                   jax.ShapeDtypeStruct((MODENUM, SLID, SLID), f32),
                   jax.ShapeDtypeStruct((MODENUM, 1, SLID), f32)),
    )(g1W, g2W, g1b, r1W, r2W, r1b)

    fwdtab_un, lin = _row_call(
        _k1_body,
        [xt, W14, p['nfW1'], p['nfb1'][None, :], p['nfW2'],
         p['nfb2'][None, :]],
        [1, None, None, None, None, None],
        [jax.ShapeDtypeStruct((MODENUM, NODENUM, 16), f32),
         jax.ShapeDtypeStruct((MODENUM, NODENUM, EMB), f32)],
        [1, 1])

    src_f = adj[:, 0, :].reshape(MODENUM * E)
    dst_f = adj[:, 1, :].reshape(MODENUM * E)
    wei_f = wei.reshape(MODENUM * E)

    dummy_tab = jnp.zeros((8, 16), f32)
    deg16 = _sc_deg(src_f, dst_f, wei_f, dummy_tab)
    deg = deg16[:, :, 0:1]

    dinv, fwdtab = _row_call(
        _k2_body,
        [deg, fwdtab_un],
        [1, 1],
        [jax.ShapeDtypeStruct((MODENUM, NODENUM, 1), f32),
         jax.ShapeDtypeStruct((MODENUM, NODENUM, 16), f32)],
        [1, 1])

    acc1 = _sc_pass16(src_f, dst_f, wei_f,
                      fwdtab.reshape(MODENUM * NODENUM, 16))

    table2, rs = _row_call(
        _k3_body,
        [acc1, fwdtab, dinv],
        [1, 1, 1],
        [jax.ShapeDtypeStruct((MODENUM, NODENUM, 16), f32),
         jax.ShapeDtypeStruct((MODENUM, NODENUM, 1), f32)],
        [1, 1])

    acc2 = _sc_pass16(src_f, dst_f, wei_f,
                      table2.reshape(MODENUM * NODENUM, 16))

    seq, rectab1, rl = _row_call(
        _k4_body,
        [acc2, table2, dinv, rs, c1, g2b, lin, cat,
         p['mlW1'], p['mlb1'][None, :], p['mlW2'], p['mlb2'][None, :],
         rWW, p['rnW1'], p['rnb1'][None, :], p['rnW2'], p['rnb2'][None, :]],
        [1, 1, 1, 1, None, None, 1, None,
         None, None, None, None, None, None, None, None, None],
        [jax.ShapeDtypeStruct((NODENUM, 2 * EMB), f32),
         jax.ShapeDtypeStruct((MODENUM, NODENUM, SLID), f32),
         jax.ShapeDtypeStruct((NODENUM, SLID), f32)],
        [0, 1, 0])

    h0 = jnp.stack([H_[0, 0], H_[1, 0]])
    b0 = jnp.stack([p['bih0'], p['bhh0']])
    b1 = jnp.stack([p['bih1'], p['bhh1']])
    y1, hT = pl.pallas_call(
        _gru_body,
        out_shape=(jax.ShapeDtypeStruct((NODENUM, GRU_H), f32),
                   jax.ShapeDtypeStruct((2, GRU_H), f32)),
        scratch_shapes=[pltpu.VMEM((NODENUM, 3 * GRU_H), f32),
                        pltpu.VMEM((NODENUM, GRU_H), f32)],
    )(seq, h0, p['Wih0'].T, p['Whh0'].T, b0, p['Wih1'].T, p['Whh1'].T, b1)

    accr1 = _sc_pass64(src_f, dst_f, wei_f,
                       rectab1.reshape(MODENUM * NODENUM, SLID))

    rectab2, = _row_call(
        _k5_body,
        [accr1, rectab1, dinv],
        [1, 1, 1],
        [jax.ShapeDtypeStruct((MODENUM, NODENUM, SLID), f32)],
        [1])

    accr2 = _sc_pass64(src_f, dst_f, wei_f,
                       rectab2.reshape(MODENUM * NODENUM, SLID))

    rec_res, = _row_call(
        _k6_body,
        [accr2, rectab2, dinv, rs, c1r, r2b, rl, p['recW'],
         p['recb'][None, :]],
        [1, 1, 1, 1, None, None, 0, None, None],
        [jax.ShapeDtypeStruct((NODENUM, SLID), f32)],
        [0])

    flat = y1.reshape(1, NODENUM * GRU_H)
    nkb = (NODENUM * GRU_H) // _CF_KB
    v = pl.pallas_call(
        _cf1_body,
        grid=(nkb,),
        in_specs=[pl.BlockSpec((1, _CF_KB), lambda i: (0, i)),
                  pl.BlockSpec((_CF_KB, 256), lambda i: (i, 0))],
        out_specs=pl.BlockSpec((1, 256), lambda i: (0, 0)),
        out_shape=jax.ShapeDtypeStruct((1, 256), f32),
        scratch_shapes=[pltpu.VMEM((1, 256), f32)],
    )(flat, p['cfW1'])

    cf_res = pl.pallas_call(
        _cf2_body,
        out_shape=jax.ShapeDtypeStruct((1, 2), f32),
    )(v, p['cfb1'][None, :], p['cfW2'], p['cfb2'][None, :],
      p['cfW3'], p['cfb3'][None, :])

    new_H = hT[:, None, :]
    return cf_res, rec_res, new_H
